# baseline (speedup 1.0000x reference)
import functools

import jax
import jax.numpy as jnp
from jax import lax
from jax.experimental import pallas as pl
from jax.experimental.pallas import tpu as pltpu
from jax.experimental.pallas import tpu_sc as plsc

N = 10000
E = 320000
IN = 128
OUT = 128
R = 8

NC = 2
NS = 16
NW = NC * NS
EW = E // NW
CH = 128
NFULL = EW // CH
TAIL = EW - NFULL * CH
NP = 10240
RPT = NP // NS

PBLK = 1000
PGRID = N // PBLK
EROW = E // 128
EBLK = EROW // PGRID

BLK = 2000
NBLK = N // BLK


def _prep_body(wc_ref, basis_ref, h_ref, typ_ref, ef_ref, wbf_ref, hbf_ref,
               gidx_ref):
    @pl.when(pl.program_id(0) == 0)
    def _():
        for r in range(R):
            w = wc_ref[r, 0] * basis_ref[0]
            for b in range(1, R):
                w = w + wc_ref[r, b] * basis_ref[b]
            wbf_ref[r] = w.astype(jnp.bfloat16)
        gidx_ref[...] = typ_ref[...] * N + ef_ref[...]

    hbf_ref[...] = h_ref[...].astype(jnp.bfloat16)


_prep = pl.pallas_call(
    _prep_body,
    grid=(PGRID,),
    out_shape=(
        jax.ShapeDtypeStruct((R, IN, OUT), jnp.bfloat16),
        jax.ShapeDtypeStruct((N, IN), jnp.bfloat16),
        jax.ShapeDtypeStruct((EROW, 128), jnp.int32),
    ),
    in_specs=[
        pl.BlockSpec(memory_space=pltpu.SMEM),
        pl.BlockSpec((R, IN, OUT), lambda nb: (0, 0, 0)),
        pl.BlockSpec((PBLK, IN), lambda nb: (nb, 0)),
        pl.BlockSpec((EROW, 128), lambda nb: (0, 0)),
        pl.BlockSpec((EROW, 128), lambda nb: (0, 0)),
    ],
    out_specs=(
        pl.BlockSpec((R, IN, OUT), lambda nb: (0, 0, 0)),
        pl.BlockSpec((PBLK, IN), lambda nb: (nb, 0)),
        pl.BlockSpec((EROW, 128), lambda nb: (0, 0)),
    ),
)


def _transform_body(h_ref, w_ref, out_ref):
    out_ref[0] = jnp.dot(
        h_ref[...], w_ref[0], preferred_element_type=jnp.float32
    )


_transform = pl.pallas_call(
    _transform_body,
    grid=(NBLK, R),
    out_shape=jax.ShapeDtypeStruct((R, N, OUT), jnp.float32),
    in_specs=[
        pl.BlockSpec((BLK, IN), lambda nb, r: (nb, 0)),
        pl.BlockSpec((1, IN, OUT), lambda nb, r: (r, 0, 0)),
    ],
    out_specs=pl.BlockSpec((1, BLK, OUT), lambda nb, r: (r, nb, 0)),
)


_mesh = plsc.VectorSubcoreMesh(
    core_axis_name="c", subcore_axis_name="s", num_cores=NC, num_subcores=NS
)


@functools.partial(
    pl.kernel,
    mesh=_mesh,
    out_type=jax.ShapeDtypeStruct((NC, NP, OUT), jnp.float32),
    scratch_types=[
        pltpu.VMEM_SHARED((NP, OUT), jnp.float32),
        pltpu.VMEM((EW,), jnp.int32),
        pltpu.VMEM((CH,), jnp.int32),
        pltpu.VMEM((CH,), jnp.int32),
        pltpu.VMEM((CH, OUT), jnp.float32),
        pltpu.VMEM((CH, OUT), jnp.float32),
        pltpu.VMEM((TAIL,), jnp.int32),
        pltpu.VMEM((TAIL, OUT), jnp.float32),
        pltpu.SemaphoreType.DMA,
        pltpu.SemaphoreType.DMA,
    ],
)
def _sc_agg(allt_hbm, edge_hbm, gidxf_hbm, out_hbm,
            agg_sh, gidx_v, ddst_a, ddst_b, rows_a, rows_b,
            tidx_v, trows_v, sem_a, sem_b):
    c = lax.axis_index("c")
    s = lax.axis_index("s")
    wid = c * NS + s
    eoff = wid * EW
    pltpu.sync_copy(gidxf_hbm.at[pl.ds(eoff, EW)], gidx_v)

    zv = jnp.zeros((16,), jnp.float32)

    def _zero(i, carry):
        for j in range(OUT // 16):
            rows_a[i, pl.ds(j * 16, 16)] = zv
        return carry

    lax.fori_loop(0, CH, _zero, 0)
    roff = s * RPT
    for i in range(RPT // CH):
        pltpu.sync_copy(rows_a, agg_sh.at[pl.ds(roff + i * CH, CH)])

    plsc.subcore_barrier()

    def _fire(k, rows_ref, ddst_ref, sem):
        pltpu.async_copy(
            allt_hbm.at[gidx_v.at[pl.ds(k * CH, CH)]], rows_ref, sem
        )
        pltpu.async_copy(
            edge_hbm.at[pl.ds(E + eoff + k * CH, CH)], ddst_ref, sem
        )

    def _wait(k, rows_ref, ddst_ref, sem):
        pltpu.make_async_copy(
            allt_hbm.at[gidx_v.at[pl.ds(k * CH, CH)]], rows_ref, sem
        ).wait()
        pltpu.make_async_copy(
            edge_hbm.at[pl.ds(E + eoff + k * CH, CH)], ddst_ref, sem
        ).wait()

    _fire(0, rows_a, ddst_a, sem_a)

    def _pair(t, carry):
        ka = 2 * t
        kb = 2 * t + 1
        _fire(kb, rows_b, ddst_b, sem_b)
        _wait(ka, rows_a, ddst_a, sem_a)
        pltpu.sync_copy(rows_a, agg_sh.at[ddst_a], add=True)

        @pl.when(t < NFULL // 2 - 1)
        def _():
            _fire(ka + 2, rows_a, ddst_a, sem_a)

        _wait(kb, rows_b, ddst_b, sem_b)
        pltpu.sync_copy(rows_b, agg_sh.at[ddst_b], add=True)
        return carry

    lax.fori_loop(0, NFULL // 2, _pair, 0)

    tbase = NFULL * CH
    cpt = pltpu.async_copy(
        allt_hbm.at[gidx_v.at[pl.ds(tbase, TAIL)]], trows_v, sem_a
    )
    pltpu.sync_copy(edge_hbm.at[pl.ds(E + eoff + tbase, TAIL)], tidx_v)
    cpt.wait()
    pltpu.sync_copy(trows_v, agg_sh.at[tidx_v], add=True)

    plsc.subcore_barrier()
    pltpu.sync_copy(agg_sh.at[pl.ds(roff, RPT)],
                    out_hbm.at[c, pl.ds(roff, RPT)])


def _final_body(p_ref, h_ref, lw_ref, b_ref, g_ref, be_ref, o_ref):
    x = (
        p_ref[0]
        + p_ref[1]
        + jnp.dot(h_ref[...], lw_ref[...], preferred_element_type=jnp.float32)
        + b_ref[...]
    )
    mu = jnp.mean(x, axis=1, keepdims=True)
    xc = x - mu
    var = jnp.mean(xc * xc, axis=1, keepdims=True)
    y = xc * lax.rsqrt(var + 1e-5) * g_ref[...] + be_ref[...]
    o_ref[...] = jnp.maximum(y, 0.0)


FBLK = 1000
FGRID = N // FBLK

_final = pl.pallas_call(
    _final_body,
    grid=(FGRID,),
    out_shape=jax.ShapeDtypeStruct((N, OUT), jnp.float32),
    in_specs=[
        pl.BlockSpec((NC, FBLK, OUT), lambda nb: (0, nb, 0)),
        pl.BlockSpec((FBLK, IN), lambda nb: (nb, 0)),
        pl.BlockSpec((IN, OUT), lambda nb: (0, 0)),
        pl.BlockSpec((1, OUT), lambda nb: (0, 0)),
        pl.BlockSpec((1, OUT), lambda nb: (0, 0)),
        pl.BlockSpec((1, OUT), lambda nb: (0, 0)),
    ],
    out_specs=pl.BlockSpec((FBLK, OUT), lambda nb: (nb, 0)),
)


def kernel(h, edge_index, edge_type, basis, w_comp, loop_weight, bias,
           ln_gamma, ln_beta):
    edge_flat = edge_index.reshape(2 * E)
    src2d = edge_flat[:E].reshape(EROW, 128)
    typ2d = edge_type.reshape(EROW, 128)
    w_bf, h_bf, gidx2d = _prep(w_comp, basis, h, typ2d, src2d)
    allt = _transform(h_bf, w_bf)
    allt2d = allt.reshape(R * N, OUT)
    parts = _sc_agg(allt2d, edge_flat, gidx2d.reshape(E))
    return _final(
        parts,
        h,
        loop_weight,
        bias.reshape(1, OUT),
        ln_gamma.reshape(1, OUT),
        ln_beta.reshape(1, OUT),
    )


# speedup vs baseline: 1.1623x; 1.1623x over previous
import functools

import jax
import jax.numpy as jnp
from jax import lax
from jax.experimental import pallas as pl
from jax.experimental.pallas import tpu as pltpu
from jax.experimental.pallas import tpu_sc as plsc

N = 10000
E = 320000
IN = 128
OUT = 128
R = 8

NC = 2
NS = 16
NW = NC * NS
EW = E // NW
CH = 128
NFULL = EW // CH
TAIL = EW - NFULL * CH
NP = 10240
RPT = NP // NS

PBLK = 2000
PGRID = N // PBLK
EROW = E // 128
EBLK = EROW // PGRID

BLK = 5000
NBLK = N // BLK


def _prep_body(wc_ref, basis_ref, h_ref, typ_ref, ef_ref, wbf_ref, hbf_ref,
               gidx_ref):
    @pl.when(pl.program_id(0) == 0)
    def _():
        for r in range(R):
            w = wc_ref[r, 0] * basis_ref[0]
            for b in range(1, R):
                w = w + wc_ref[r, b] * basis_ref[b]
            wbf_ref[r] = w.astype(jnp.bfloat16)
        gidx_ref[...] = typ_ref[...] * N + ef_ref[0]

    hbf_ref[...] = h_ref[...].astype(jnp.bfloat16)


_prep = pl.pallas_call(
    _prep_body,
    grid=(PGRID,),
    out_shape=(
        jax.ShapeDtypeStruct((R, IN, OUT), jnp.bfloat16),
        jax.ShapeDtypeStruct((N, IN), jnp.bfloat16),
        jax.ShapeDtypeStruct((EROW, 128), jnp.int32),
    ),
    in_specs=[
        pl.BlockSpec(memory_space=pltpu.SMEM),
        pl.BlockSpec((R, IN, OUT), lambda nb: (0, 0, 0)),
        pl.BlockSpec((PBLK, IN), lambda nb: (nb, 0)),
        pl.BlockSpec((EROW, 128), lambda nb: (0, 0)),
        pl.BlockSpec((1, EROW, 128), lambda nb: (0, 0, 0)),
    ],
    out_specs=(
        pl.BlockSpec((R, IN, OUT), lambda nb: (0, 0, 0)),
        pl.BlockSpec((PBLK, IN), lambda nb: (nb, 0)),
        pl.BlockSpec((EROW, 128), lambda nb: (0, 0)),
    ),
)


def _transform_body(h_ref, w_ref, out_ref):
    out_ref[...] = jnp.dot(
        h_ref[...], w_ref[0], preferred_element_type=jnp.float32
    )


_transform = pl.pallas_call(
    _transform_body,
    grid=(NBLK, R),
    out_shape=jax.ShapeDtypeStruct((R * N, OUT), jnp.float32),
    in_specs=[
        pl.BlockSpec((BLK, IN), lambda nb, r: (nb, 0)),
        pl.BlockSpec((1, IN, OUT), lambda nb, r: (r, 0, 0)),
    ],
    out_specs=pl.BlockSpec((BLK, OUT), lambda nb, r: (r * NBLK + nb, 0)),
)


_mesh = plsc.VectorSubcoreMesh(
    core_axis_name="c", subcore_axis_name="s", num_cores=NC, num_subcores=NS
)


@functools.partial(
    pl.kernel,
    mesh=_mesh,
    out_type=jax.ShapeDtypeStruct((NC, NP, OUT), jnp.float32),
    scratch_types=[
        pltpu.VMEM_SHARED((NP, OUT), jnp.float32),
        pltpu.VMEM((EW,), jnp.int32),
        pltpu.VMEM((CH,), jnp.int32),
        pltpu.VMEM((CH,), jnp.int32),
        pltpu.VMEM((CH, OUT), jnp.float32),
        pltpu.VMEM((CH, OUT), jnp.float32),
        pltpu.VMEM((TAIL,), jnp.int32),
        pltpu.VMEM((TAIL, OUT), jnp.float32),
        pltpu.SemaphoreType.DMA,
        pltpu.SemaphoreType.DMA,
    ],
)
def _sc_agg(allt_hbm, edge_hbm, gidxf_hbm, out_hbm,
            agg_sh, gidx_v, ddst_a, ddst_b, rows_a, rows_b,
            tidx_v, trows_v, sem_a, sem_b):
    c = lax.axis_index("c")
    s = lax.axis_index("s")
    wid = c * NS + s
    eoff = wid * EW
    pltpu.sync_copy(gidxf_hbm.at[pl.ds(eoff, EW)], gidx_v)

    zv = jnp.zeros((16,), jnp.float32)

    def _zero(i, carry):
        for j in range(OUT // 16):
            rows_a[i, pl.ds(j * 16, 16)] = zv
        return carry

    lax.fori_loop(0, CH, _zero, 0)
    roff = s * RPT
    for i in range(RPT // CH):
        pltpu.sync_copy(rows_a, agg_sh.at[pl.ds(roff + i * CH, CH)])

    plsc.subcore_barrier()

    def _fire(k, rows_ref, ddst_ref, sem):
        pltpu.async_copy(
            allt_hbm.at[gidx_v.at[pl.ds(k * CH, CH)]], rows_ref, sem
        )
        pltpu.async_copy(
            edge_hbm.at[pl.ds(E + eoff + k * CH, CH)], ddst_ref, sem
        )

    def _wait(k, rows_ref, ddst_ref, sem):
        pltpu.make_async_copy(
            allt_hbm.at[gidx_v.at[pl.ds(k * CH, CH)]], rows_ref, sem
        ).wait()
        pltpu.make_async_copy(
            edge_hbm.at[pl.ds(E + eoff + k * CH, CH)], ddst_ref, sem
        ).wait()

    _fire(0, rows_a, ddst_a, sem_a)

    def _pair(t, carry):
        ka = 2 * t
        kb = 2 * t + 1
        _fire(kb, rows_b, ddst_b, sem_b)
        _wait(ka, rows_a, ddst_a, sem_a)
        pltpu.sync_copy(rows_a, agg_sh.at[ddst_a], add=True)

        @pl.when(t < NFULL // 2 - 1)
        def _():
            _fire(ka + 2, rows_a, ddst_a, sem_a)

        _wait(kb, rows_b, ddst_b, sem_b)
        pltpu.sync_copy(rows_b, agg_sh.at[ddst_b], add=True)
        return carry

    lax.fori_loop(0, NFULL // 2, _pair, 0)

    tbase = NFULL * CH
    cpt = pltpu.async_copy(
        allt_hbm.at[gidx_v.at[pl.ds(tbase, TAIL)]], trows_v, sem_a
    )
    pltpu.sync_copy(edge_hbm.at[pl.ds(E + eoff + tbase, TAIL)], tidx_v)
    cpt.wait()
    pltpu.sync_copy(trows_v, agg_sh.at[tidx_v], add=True)

    plsc.subcore_barrier()
    pltpu.sync_copy(agg_sh.at[pl.ds(roff, RPT)],
                    out_hbm.at[c, pl.ds(roff, RPT)])


def _final_body(p_ref, h_ref, lw_ref, b_ref, g_ref, be_ref, o_ref):
    x = (
        p_ref[0]
        + p_ref[1]
        + jnp.dot(h_ref[...], lw_ref[...], preferred_element_type=jnp.float32)
        + b_ref[...]
    )
    mu = jnp.mean(x, axis=1, keepdims=True)
    xc = x - mu
    var = jnp.mean(xc * xc, axis=1, keepdims=True)
    y = xc * lax.rsqrt(var + 1e-5) * g_ref[...] + be_ref[...]
    o_ref[...] = jnp.maximum(y, 0.0)


FBLK = 2000
FGRID = N // FBLK

_final = pl.pallas_call(
    _final_body,
    grid=(FGRID,),
    out_shape=jax.ShapeDtypeStruct((N, OUT), jnp.float32),
    in_specs=[
        pl.BlockSpec((NC, FBLK, OUT), lambda nb: (0, nb, 0)),
        pl.BlockSpec((FBLK, IN), lambda nb: (nb, 0)),
        pl.BlockSpec((IN, OUT), lambda nb: (0, 0)),
        pl.BlockSpec((1, OUT), lambda nb: (0, 0)),
        pl.BlockSpec((1, OUT), lambda nb: (0, 0)),
        pl.BlockSpec((1, OUT), lambda nb: (0, 0)),
    ],
    out_specs=pl.BlockSpec((FBLK, OUT), lambda nb: (nb, 0)),
)


def kernel(h, edge_index, edge_type, basis, w_comp, loop_weight, bias,
           ln_gamma, ln_beta):
    edge_flat = edge_index.reshape(2 * E)
    src2d = edge_flat.reshape(2, EROW, 128)
    typ2d = edge_type.reshape(EROW, 128)
    w_bf, h_bf, gidx2d = _prep(w_comp, basis, h, typ2d, src2d)
    allt2d = _transform(h_bf, w_bf)
    parts = _sc_agg(allt2d, edge_flat, gidx2d.reshape(E))
    return _final(
        parts,
        h,
        loop_weight,
        bias.reshape(1, OUT),
        ln_gamma.reshape(1, OUT),
        ln_beta.reshape(1, OUT),
    )


# speedup vs baseline: 1.2343x; 1.0619x over previous
import functools

import jax
import jax.numpy as jnp
from jax import lax
from jax.experimental import pallas as pl
from jax.experimental.pallas import tpu as pltpu
from jax.experimental.pallas import tpu_sc as plsc

N = 10000
E = 320000
IN = 128
OUT = 128
R = 8

NC = 2
NS = 16
NW = NC * NS
EW = E // NW
CH = 128
NFULL = EW // CH
TAIL = EW - NFULL * CH
NP = 10240
RPT = NP // NS

PBLK = 2000
PGRID = N // PBLK
EROW = E // 128
EBLK = EROW // PGRID

BLK = 5000
NBLK = N // BLK


def _transform_body(wc_ref, basis_ref, h_ref, typ_ref, ef_ref,
                    out_ref, gidx_ref, hbf_ref):
    r = pl.program_id(0)

    @pl.when(r == 0)
    def _():
        hbf_ref[...] = h_ref[...].astype(jnp.bfloat16)
        gidx_ref[...] = typ_ref[...] * N + ef_ref[0]

    w = wc_ref[r, 0] * basis_ref[0]
    for b in range(1, R):
        w = w + wc_ref[r, b] * basis_ref[b]
    out_ref[...] = jnp.dot(
        hbf_ref[...], w.astype(jnp.bfloat16), preferred_element_type=jnp.float32
    )


_transform = pl.pallas_call(
    _transform_body,
    grid=(R,),
    out_shape=(
        jax.ShapeDtypeStruct((R * N, OUT), jnp.float32),
        jax.ShapeDtypeStruct((EROW, 128), jnp.int32),
    ),
    in_specs=[
        pl.BlockSpec(memory_space=pltpu.SMEM),
        pl.BlockSpec((R, IN, OUT), lambda r: (0, 0, 0)),
        pl.BlockSpec((N, IN), lambda r: (0, 0)),
        pl.BlockSpec((EROW, 128), lambda r: (0, 0)),
        pl.BlockSpec((1, EROW, 128), lambda r: (0, 0, 0)),
    ],
    out_specs=(
        pl.BlockSpec((N, OUT), lambda r: (r, 0)),
        pl.BlockSpec((EROW, 128), lambda r: (0, 0)),
    ),
    scratch_shapes=[pltpu.VMEM((N, IN), jnp.bfloat16)],
)


_mesh = plsc.VectorSubcoreMesh(
    core_axis_name="c", subcore_axis_name="s", num_cores=NC, num_subcores=NS
)


@functools.partial(
    pl.kernel,
    mesh=_mesh,
    out_type=jax.ShapeDtypeStruct((NC, NP, OUT), jnp.float32),
    scratch_types=[
        pltpu.VMEM_SHARED((NP, OUT), jnp.float32),
        pltpu.VMEM((EW,), jnp.int32),
        pltpu.VMEM((CH,), jnp.int32),
        pltpu.VMEM((CH,), jnp.int32),
        pltpu.VMEM((CH, OUT), jnp.float32),
        pltpu.VMEM((CH, OUT), jnp.float32),
        pltpu.VMEM((TAIL,), jnp.int32),
        pltpu.VMEM((TAIL, OUT), jnp.float32),
        pltpu.SemaphoreType.DMA,
        pltpu.SemaphoreType.DMA,
    ],
)
def _sc_agg(allt_hbm, edge_hbm, gidxf_hbm, out_hbm,
            agg_sh, gidx_v, ddst_a, ddst_b, rows_a, rows_b,
            tidx_v, trows_v, sem_a, sem_b):
    c = lax.axis_index("c")
    s = lax.axis_index("s")
    wid = c * NS + s
    eoff = wid * EW
    pltpu.sync_copy(gidxf_hbm.at[pl.ds(eoff, EW)], gidx_v)

    def _fire(k, rows_ref, ddst_ref, sem):
        pltpu.async_copy(
            allt_hbm.at[gidx_v.at[pl.ds(k * CH, CH)]], rows_ref, sem
        )
        pltpu.async_copy(
            edge_hbm.at[pl.ds(E + eoff + k * CH, CH)], ddst_ref, sem
        )

    def _wait(k, rows_ref, ddst_ref, sem):
        pltpu.make_async_copy(
            allt_hbm.at[gidx_v.at[pl.ds(k * CH, CH)]], rows_ref, sem
        ).wait()
        pltpu.make_async_copy(
            edge_hbm.at[pl.ds(E + eoff + k * CH, CH)], ddst_ref, sem
        ).wait()

    zv = jnp.zeros((16,), jnp.float32)

    def _zero(i, carry):
        for j in range(OUT // 16):
            rows_a[i, pl.ds(j * 16, 16)] = zv
        return carry

    lax.fori_loop(0, CH, _zero, 0)
    roff = s * RPT
    for i in range(RPT // CH):
        pltpu.sync_copy(rows_a, agg_sh.at[pl.ds(roff + i * CH, CH)])

    _fire(0, rows_a, ddst_a, sem_a)
    _fire(1, rows_b, ddst_b, sem_b)

    plsc.subcore_barrier()

    def _pair(t, carry):
        ka = 2 * t
        kb = 2 * t + 1
        _wait(ka, rows_a, ddst_a, sem_a)
        pltpu.sync_copy(rows_a, agg_sh.at[ddst_a], add=True)

        @pl.when(t < NFULL // 2 - 1)
        def _():
            _fire(ka + 2, rows_a, ddst_a, sem_a)

        _wait(kb, rows_b, ddst_b, sem_b)
        pltpu.sync_copy(rows_b, agg_sh.at[ddst_b], add=True)

        @pl.when(t < NFULL // 2 - 1)
        def _():
            _fire(kb + 2, rows_b, ddst_b, sem_b)

        return carry

    lax.fori_loop(0, NFULL // 2, _pair, 0)

    tbase = NFULL * CH
    cpt = pltpu.async_copy(
        allt_hbm.at[gidx_v.at[pl.ds(tbase, TAIL)]], trows_v, sem_a
    )
    pltpu.sync_copy(edge_hbm.at[pl.ds(E + eoff + tbase, TAIL)], tidx_v)
    cpt.wait()
    pltpu.sync_copy(trows_v, agg_sh.at[tidx_v], add=True)

    plsc.subcore_barrier()
    pltpu.sync_copy(agg_sh.at[pl.ds(roff, RPT)],
                    out_hbm.at[c, pl.ds(roff, RPT)])


def _final_body(p_ref, h_ref, lw_ref, b_ref, g_ref, be_ref, o_ref):
    x = (
        p_ref[0]
        + p_ref[1]
        + jnp.dot(h_ref[...], lw_ref[...], preferred_element_type=jnp.float32)
        + b_ref[...]
    )
    mu = jnp.mean(x, axis=1, keepdims=True)
    xc = x - mu
    var = jnp.mean(xc * xc, axis=1, keepdims=True)
    y = xc * lax.rsqrt(var + 1e-5) * g_ref[...] + be_ref[...]
    o_ref[...] = jnp.maximum(y, 0.0)


FBLK = 2000
FGRID = N // FBLK

_final = pl.pallas_call(
    _final_body,
    grid=(FGRID,),
    out_shape=jax.ShapeDtypeStruct((N, OUT), jnp.float32),
    in_specs=[
        pl.BlockSpec((NC, FBLK, OUT), lambda nb: (0, nb, 0)),
        pl.BlockSpec((FBLK, IN), lambda nb: (nb, 0)),
        pl.BlockSpec((IN, OUT), lambda nb: (0, 0)),
        pl.BlockSpec((1, OUT), lambda nb: (0, 0)),
        pl.BlockSpec((1, OUT), lambda nb: (0, 0)),
        pl.BlockSpec((1, OUT), lambda nb: (0, 0)),
    ],
    out_specs=pl.BlockSpec((FBLK, OUT), lambda nb: (nb, 0)),
)


def kernel(h, edge_index, edge_type, basis, w_comp, loop_weight, bias,
           ln_gamma, ln_beta):
    edge_flat = edge_index.reshape(2 * E)
    src2d = edge_flat.reshape(2, EROW, 128)
    typ2d = edge_type.reshape(EROW, 128)
    allt2d, gidx2d = _transform(w_comp, basis, h, typ2d, src2d)
    parts = _sc_agg(allt2d, edge_flat, gidx2d.reshape(E))
    return _final(
        parts,
        h,
        loop_weight,
        bias.reshape(1, OUT),
        ln_gamma.reshape(1, OUT),
        ln_beta.reshape(1, OUT),
    )


# speedup vs baseline: 1.2441x; 1.0080x over previous
import functools

import jax
import jax.numpy as jnp
from jax import lax
from jax.experimental import pallas as pl
from jax.experimental.pallas import tpu as pltpu
from jax.experimental.pallas import tpu_sc as plsc

N = 10000
E = 320000
IN = 128
OUT = 128
R = 8

NC = 2
NS = 16
NW = NC * NS
EW = E // NW
CH = 128
NFULL = EW // CH
TAIL = EW - NFULL * CH
NP = 10240
RPT = NP // NS

PBLK = 2000
PGRID = N // PBLK
EROW = E // 128
EBLK = EROW // PGRID

BLK = 5000
NBLK = N // BLK


def _transform_body(wc_ref, basis_ref, h_ref, typ_ref, ef_ref,
                    out_ref, gidx_ref, hbf_ref):
    r = pl.program_id(0)

    @pl.when(r == 0)
    def _():
        hbf_ref[...] = h_ref[...].astype(jnp.bfloat16)
        gidx_ref[...] = typ_ref[...] * N + ef_ref[0]

    w = wc_ref[r, 0] * basis_ref[0]
    for b in range(1, R):
        w = w + wc_ref[r, b] * basis_ref[b]
    out_ref[...] = jnp.dot(
        hbf_ref[...], w.astype(jnp.bfloat16), preferred_element_type=jnp.float32
    )


_transform = pl.pallas_call(
    _transform_body,
    grid=(R,),
    out_shape=(
        jax.ShapeDtypeStruct((R * N, OUT), jnp.float32),
        jax.ShapeDtypeStruct((EROW, 128), jnp.int32),
    ),
    in_specs=[
        pl.BlockSpec(memory_space=pltpu.SMEM),
        pl.BlockSpec((R, IN, OUT), lambda r: (0, 0, 0)),
        pl.BlockSpec((N, IN), lambda r: (0, 0)),
        pl.BlockSpec((EROW, 128), lambda r: (0, 0)),
        pl.BlockSpec((1, EROW, 128), lambda r: (0, 0, 0)),
    ],
    out_specs=(
        pl.BlockSpec((N, OUT), lambda r: (r, 0)),
        pl.BlockSpec((EROW, 128), lambda r: (0, 0)),
    ),
    scratch_shapes=[pltpu.VMEM((N, IN), jnp.bfloat16)],
)


_mesh = plsc.VectorSubcoreMesh(
    core_axis_name="c", subcore_axis_name="s", num_cores=NC, num_subcores=NS
)


@functools.partial(
    pl.kernel,
    mesh=_mesh,
    out_type=jax.ShapeDtypeStruct((NC, NP, OUT), jnp.float32),
    scratch_types=[
        pltpu.VMEM_SHARED((NP, OUT), jnp.float32),
        pltpu.VMEM((EW,), jnp.int32),
        pltpu.VMEM((CH,), jnp.int32),
        pltpu.VMEM((CH,), jnp.int32),
        pltpu.VMEM((CH, OUT), jnp.float32),
        pltpu.VMEM((CH, OUT), jnp.float32),
        pltpu.VMEM((TAIL,), jnp.int32),
        pltpu.VMEM((TAIL, OUT), jnp.float32),
        pltpu.SemaphoreType.DMA,
        pltpu.SemaphoreType.DMA,
    ],
)
def _sc_agg(allt_hbm, edge_hbm, gidxf_hbm, out_hbm,
            agg_sh, gidx_v, ddst_a, ddst_b, rows_a, rows_b,
            tidx_v, trows_v, sem_a, sem_b):
    c = lax.axis_index("c")
    s = lax.axis_index("s")
    wid = c * NS + s
    eoff = wid * EW
    pltpu.sync_copy(gidxf_hbm.at[pl.ds(eoff, EW)], gidx_v)

    def _fire(k, rows_ref, ddst_ref, sem):
        pltpu.async_copy(
            allt_hbm.at[gidx_v.at[pl.ds(k * CH, CH)]], rows_ref, sem
        )
        pltpu.async_copy(
            edge_hbm.at[pl.ds(E + eoff + k * CH, CH)], ddst_ref, sem
        )

    def _wait(k, rows_ref, ddst_ref, sem):
        pltpu.make_async_copy(
            allt_hbm.at[gidx_v.at[pl.ds(k * CH, CH)]], rows_ref, sem
        ).wait()
        pltpu.make_async_copy(
            edge_hbm.at[pl.ds(E + eoff + k * CH, CH)], ddst_ref, sem
        ).wait()

    zv = jnp.zeros((16,), jnp.float32)

    def _zero(i, carry):
        for j in range(OUT // 16):
            rows_a[i, pl.ds(j * 16, 16)] = zv
        return carry

    lax.fori_loop(0, CH, _zero, 0)
    roff = s * RPT
    for i in range(RPT // CH):
        pltpu.sync_copy(rows_a, agg_sh.at[pl.ds(roff + i * CH, CH)])

    _fire(0, rows_a, ddst_a, sem_a)
    _fire(1, rows_b, ddst_b, sem_b)

    plsc.subcore_barrier()

    def _pair(t, carry):
        ka = 2 * t
        kb = 2 * t + 1
        _wait(ka, rows_a, ddst_a, sem_a)
        pltpu.sync_copy(rows_a, agg_sh.at[ddst_a], add=True)

        @pl.when(t < NFULL // 2 - 1)
        def _():
            _fire(ka + 2, rows_a, ddst_a, sem_a)

        _wait(kb, rows_b, ddst_b, sem_b)
        pltpu.sync_copy(rows_b, agg_sh.at[ddst_b], add=True)

        @pl.when(t < NFULL // 2 - 1)
        def _():
            _fire(kb + 2, rows_b, ddst_b, sem_b)

        return carry

    lax.fori_loop(0, NFULL // 2, _pair, 0)

    tbase = NFULL * CH
    cpt = pltpu.async_copy(
        allt_hbm.at[gidx_v.at[pl.ds(tbase, TAIL)]], trows_v, sem_a
    )
    pltpu.sync_copy(edge_hbm.at[pl.ds(E + eoff + tbase, TAIL)], tidx_v)
    cpt.wait()
    pltpu.sync_copy(trows_v, agg_sh.at[tidx_v], add=True)

    plsc.subcore_barrier()
    pltpu.sync_copy(agg_sh.at[pl.ds(roff, RPT)],
                    out_hbm.at[c, pl.ds(roff, RPT)])


def _selfloop_body(h_ref, lw_ref, o_ref):
    o_ref[...] = jnp.dot(
        h_ref[...], lw_ref[...], preferred_element_type=jnp.float32
    )


SBLK = 5000

_selfloop = pl.pallas_call(
    _selfloop_body,
    grid=(N // SBLK,),
    out_shape=jax.ShapeDtypeStruct((N, OUT), jnp.float32),
    in_specs=[
        pl.BlockSpec((SBLK, IN), lambda nb: (nb, 0)),
        pl.BlockSpec((IN, OUT), lambda nb: (0, 0)),
    ],
    out_specs=pl.BlockSpec((SBLK, OUT), lambda nb: (nb, 0)),
)


def _final_body(p_ref, sl_ref, b_ref, g_ref, be_ref, o_ref):
    x = p_ref[0] + p_ref[1] + sl_ref[...] + b_ref[...]
    mu = jnp.mean(x, axis=1, keepdims=True)
    xc = x - mu
    var = jnp.mean(xc * xc, axis=1, keepdims=True)
    y = xc * lax.rsqrt(var + 1e-5) * g_ref[...] + be_ref[...]
    o_ref[...] = jnp.maximum(y, 0.0)


FBLK = 5000
FGRID = N // FBLK

_final = pl.pallas_call(
    _final_body,
    grid=(FGRID,),
    out_shape=jax.ShapeDtypeStruct((N, OUT), jnp.float32),
    in_specs=[
        pl.BlockSpec((NC, FBLK, OUT), lambda nb: (0, nb, 0)),
        pl.BlockSpec((FBLK, OUT), lambda nb: (nb, 0)),
        pl.BlockSpec((1, OUT), lambda nb: (0, 0)),
        pl.BlockSpec((1, OUT), lambda nb: (0, 0)),
        pl.BlockSpec((1, OUT), lambda nb: (0, 0)),
    ],
    out_specs=pl.BlockSpec((FBLK, OUT), lambda nb: (nb, 0)),
)


def kernel(h, edge_index, edge_type, basis, w_comp, loop_weight, bias,
           ln_gamma, ln_beta):
    edge_flat = edge_index.reshape(2 * E)
    src2d = edge_flat.reshape(2, EROW, 128)
    typ2d = edge_type.reshape(EROW, 128)
    allt2d, gidx2d = _transform(w_comp, basis, h, typ2d, src2d)
    selfloop = _selfloop(h, loop_weight)
    parts = _sc_agg(allt2d, edge_flat, gidx2d.reshape(E))
    return _final(
        parts,
        selfloop,
        bias.reshape(1, OUT),
        ln_gamma.reshape(1, OUT),
        ln_beta.reshape(1, OUT),
    )
